# baseline (speedup 1.0000x reference)
import functools

import jax
import jax.numpy as jnp
from jax import lax
from jax.experimental import pallas as pl
from jax.experimental.pallas import tpu as pltpu
from jax.experimental.pallas import tpu_sc as plsc

_DIM = 768
_E = 64
_K = 8
_N = 32768
_T = 256
_NB = _N // _T

_NC = 2
_NS = 16
_NW = _NC * _NS
_S = _N * _K // _NW
_COLS = 128
_ROWS = _S // _COLS


def _tc_body(x_ref, bias_ref, wt_ref, vals_ref, sel_ref, posng_ref,
             counts_ref, offs_ref, carry, ltri):
    pid = pl.program_id(0)

    @pl.when(pid == 0)
    def _init():
        carry[...] = jnp.zeros_like(carry)
        r = lax.broadcasted_iota(jnp.int32, (_T, _T), 0)
        c = lax.broadcasted_iota(jnp.int32, (_T, _T), 1)
        ltri[...] = (c < r).astype(jnp.float32)

    x = x_ref[...]
    wt = wt_ref[...]
    logits = jnp.dot(x, wt, preferred_element_type=jnp.float32)
    m = jnp.max(logits, axis=1, keepdims=True)
    ex = jnp.exp(logits - m)
    probs = ex / jnp.sum(ex, axis=1, keepdims=True)

    bias = bias_ref[0:1, :]
    work = probs + bias
    lane = lax.broadcasted_iota(jnp.int32, (_T, _E), 1)
    hist = jnp.zeros((_T, _E), jnp.float32)
    sel_cols = []
    val_cols = []
    for _ in range(_K):
        mk = jnp.max(work, axis=1, keepdims=True)
        cand = jnp.where(work == mk, lane, _E)
        idx = jnp.min(cand, axis=1, keepdims=True)
        hot = lane == idx
        sel_cols.append(idx)
        val_cols.append(jnp.sum(jnp.where(hot, probs, 0.0), axis=1,
                                keepdims=True))
        work = jnp.where(hot, -jnp.inf, work)
        hist = hist + hot.astype(jnp.float32)

    vals_ref[...] = jnp.concatenate(val_cols, axis=1)
    sel_ref[...] = jnp.concatenate(sel_cols, axis=1)

    cum = jnp.dot(ltri[...], hist, preferred_element_type=jnp.float32)
    pos_tok = carry[0:1, :] + cum
    png_cols = [jnp.sum(jnp.where(lane == s, pos_tok, 0.0), axis=1,
                        keepdims=True) for s in sel_cols]
    posng_ref[...] = jnp.concatenate(png_cols, axis=1).astype(jnp.int32)

    new_carry = carry[0:1, :] + jnp.sum(hist, axis=0, keepdims=True)
    carry[...] = jnp.broadcast_to(new_carry, carry.shape)

    counts_ref[...] = jnp.broadcast_to(new_carry, (8, _E)).astype(jnp.int32)
    lo = lax.broadcasted_iota(jnp.int32, (_E, _E), 0)
    hi = lax.broadcasted_iota(jnp.int32, (_E, _E), 1)
    strict = (lo < hi).astype(jnp.float32)
    offs = jnp.dot(new_carry, strict, preferred_element_type=jnp.float32,
                   precision=lax.Precision.HIGHEST)
    offs_ref[...] = jnp.broadcast_to(offs, (8, _E)).astype(jnp.int32)


def _tc_route(x, bias8, wt):
    return pl.pallas_call(
        _tc_body,
        grid=(_NB,),
        in_specs=[
            pl.BlockSpec((_T, _DIM), lambda i: (i, 0)),
            pl.BlockSpec((8, _E), lambda i: (0, 0)),
            pl.BlockSpec((_DIM, _E), lambda i: (0, 0)),
        ],
        out_specs=[
            pl.BlockSpec((_T, _K), lambda i: (i, 0)),
            pl.BlockSpec((_T, _K), lambda i: (i, 0)),
            pl.BlockSpec((_T, _K), lambda i: (i, 0)),
            pl.BlockSpec((8, _E), lambda i: (0, 0)),
            pl.BlockSpec((8, _E), lambda i: (0, 0)),
        ],
        out_shape=[
            jax.ShapeDtypeStruct((_N, _K), jnp.float32),
            jax.ShapeDtypeStruct((_N, _K), jnp.int32),
            jax.ShapeDtypeStruct((_N, _K), jnp.int32),
            jax.ShapeDtypeStruct((8, _E), jnp.int32),
            jax.ShapeDtypeStruct((8, _E), jnp.int32),
        ],
        scratch_shapes=[
            pltpu.VMEM((8, _E), jnp.float32),
            pltpu.VMEM((_T, _T), jnp.float32),
        ],
        compiler_params=pltpu.CompilerParams(
            dimension_semantics=("arbitrary",)),
    )(x, bias8, wt)


@functools.lru_cache(maxsize=1)
def _sc_scatter_kernel():
    mesh = plsc.VectorSubcoreMesh(core_axis_name="c", subcore_axis_name="s",
                                  num_cores=_NC, num_subcores=_NS)
    return functools.partial(
        pl.kernel,
        mesh=mesh,
        out_type=[
            jax.ShapeDtypeStruct((_N * _K,), jnp.float32),
            jax.ShapeDtypeStruct((_N * _K,), jnp.int32),
        ],
        scratch_types=[
            pltpu.VMEM((_ROWS, _COLS), jnp.float32),
            pltpu.VMEM((_ROWS, _COLS), jnp.int32),
            pltpu.VMEM((_ROWS, _COLS), jnp.int32),
            pltpu.VMEM((_ROWS, _COLS), jnp.int32),
            pltpu.VMEM((_E,), jnp.int32),
            pltpu.SemaphoreType.DMA,
            pltpu.SemaphoreType.DMA,
        ],
        compiler_params=pltpu.CompilerParams(needs_layout_passes=False),
    )(_sc_scatter_body)


def _sc_scatter_body(vals_hbm, sel_hbm, posng_hbm, offs_hbm,
                out_s_hbm, out_t_hbm,
                val_v, sel_v, pos_v, tok_v, offs_v, sem_s, sem_t):
    wid = lax.axis_index("s") * _NC + lax.axis_index("c")
    row0 = wid * _ROWS
    base = wid * _S

    pltpu.sync_copy(vals_hbm.at[pl.ds(row0, _ROWS), :], val_v)
    pltpu.sync_copy(sel_hbm.at[pl.ds(row0, _ROWS), :], sel_v)
    pltpu.sync_copy(posng_hbm.at[pl.ds(row0, _ROWS), :], pos_v)
    pltpu.sync_copy(offs_hbm, offs_v)

    lane16 = lax.iota(jnp.int32, 16)

    def _chunk(j, _):
        r = j // 8
        c16 = (j % 8) * 16
        e = sel_v[r, pl.ds(c16, 16)]
        off = plsc.load_gather(offs_v, [e])
        pos_v[r, pl.ds(c16, 16)] = pos_v[r, pl.ds(c16, 16)] + off
        flat = base + r * _COLS + c16 + lane16
        tok_v[r, pl.ds(c16, 16)] = lax.shift_right_logical(flat, 3)
        return 0

    lax.fori_loop(0, _ROWS * 8, _chunk, 0)

    def _scat(j, _):
        pltpu.async_copy(val_v.at[j], out_s_hbm.at[pos_v.at[j]], sem_s).wait()
        pltpu.async_copy(tok_v.at[j], out_t_hbm.at[pos_v.at[j]], sem_t).wait()
        return 0

    lax.fori_loop(0, _ROWS, _scat, 0)


def kernel(x, expert_bias, W):
    bias8 = jnp.broadcast_to(expert_bias.reshape(1, _E), (8, _E))
    wt = W.T
    vals, sel, posng, counts8, offs8 = _tc_route(x, bias8, wt)
    vals2 = vals.reshape(_NW * _ROWS, _COLS)
    sel2 = sel.reshape(_NW * _ROWS, _COLS)
    posng2 = posng.reshape(_NW * _ROWS, _COLS)
    offs = offs8[0]
    out_s, out_t = _sc_scatter_kernel()(vals2, sel2, posng2, offs)
    return out_s, out_t, counts8[0]


# speedup vs baseline: 1.0030x; 1.0030x over previous
import functools

import jax
import jax.numpy as jnp
from jax import lax
from jax.experimental import pallas as pl
from jax.experimental.pallas import tpu as pltpu
from jax.experimental.pallas import tpu_sc as plsc

_DIM = 768
_E = 64
_K = 8
_N = 32768
_T = 256
_NB = _N // _T

_NC = 2
_NS = 16
_NW = _NC * _NS
_S = _N * _K // _NW
_COLS = 128
_ROWS = _S // _COLS


def _tc_body(x_ref, bias_ref, wt_ref, vals_ref, sel_ref, posng_ref,
             counts_ref, offs_ref, carry, ltri):
    pid = pl.program_id(0)

    @pl.when(pid == 0)
    def _init():
        carry[...] = jnp.zeros_like(carry)
        r = lax.broadcasted_iota(jnp.int32, (_T, _T), 0)
        c = lax.broadcasted_iota(jnp.int32, (_T, _T), 1)
        ltri[...] = (c < r).astype(jnp.float32)

    x = x_ref[...]
    wt = wt_ref[...]
    logits = jnp.dot(x, wt, preferred_element_type=jnp.float32)
    m = jnp.max(logits, axis=1, keepdims=True)
    ex = jnp.exp(logits - m)
    probs = ex / jnp.sum(ex, axis=1, keepdims=True)

    bias = bias_ref[0:1, :]
    work = probs + bias
    lane = lax.broadcasted_iota(jnp.int32, (_T, _E), 1)
    hist = jnp.zeros((_T, _E), jnp.float32)
    sel_cols = []
    val_cols = []
    for _ in range(_K):
        mk = jnp.max(work, axis=1, keepdims=True)
        cand = jnp.where(work == mk, lane, _E)
        idx = jnp.min(cand, axis=1, keepdims=True)
        hot = lane == idx
        sel_cols.append(idx)
        val_cols.append(jnp.sum(jnp.where(hot, probs, 0.0), axis=1,
                                keepdims=True))
        work = jnp.where(hot, -jnp.inf, work)
        hist = hist + hot.astype(jnp.float32)

    vals_ref[...] = jnp.concatenate(val_cols, axis=1)
    sel_ref[...] = jnp.concatenate(sel_cols, axis=1)

    cum = jnp.dot(ltri[...], hist, preferred_element_type=jnp.float32)
    pos_tok = carry[0:1, :] + cum
    png_cols = [jnp.sum(jnp.where(lane == s, pos_tok, 0.0), axis=1,
                        keepdims=True) for s in sel_cols]
    posng_ref[...] = jnp.concatenate(png_cols, axis=1).astype(jnp.int32)

    new_carry = carry[0:1, :] + jnp.sum(hist, axis=0, keepdims=True)
    carry[...] = jnp.broadcast_to(new_carry, carry.shape)

    counts_ref[...] = jnp.broadcast_to(new_carry, (8, _E)).astype(jnp.int32)
    lo = lax.broadcasted_iota(jnp.int32, (_E, _E), 0)
    hi = lax.broadcasted_iota(jnp.int32, (_E, _E), 1)
    strict = (lo < hi).astype(jnp.float32)
    offs = jnp.dot(new_carry, strict, preferred_element_type=jnp.float32,
                   precision=lax.Precision.HIGHEST)
    offs_ref[...] = jnp.broadcast_to(offs, (8, _E)).astype(jnp.int32)


def _tc_route(x, bias8, wt):
    return pl.pallas_call(
        _tc_body,
        grid=(_NB,),
        in_specs=[
            pl.BlockSpec((_T, _DIM), lambda i: (i, 0)),
            pl.BlockSpec((8, _E), lambda i: (0, 0)),
            pl.BlockSpec((_DIM, _E), lambda i: (0, 0)),
        ],
        out_specs=[
            pl.BlockSpec((_T, _K), lambda i: (i, 0)),
            pl.BlockSpec((_T, _K), lambda i: (i, 0)),
            pl.BlockSpec((_T, _K), lambda i: (i, 0)),
            pl.BlockSpec((8, _E), lambda i: (0, 0)),
            pl.BlockSpec((8, _E), lambda i: (0, 0)),
        ],
        out_shape=[
            jax.ShapeDtypeStruct((_N, _K), jnp.float32),
            jax.ShapeDtypeStruct((_N, _K), jnp.int32),
            jax.ShapeDtypeStruct((_N, _K), jnp.int32),
            jax.ShapeDtypeStruct((8, _E), jnp.int32),
            jax.ShapeDtypeStruct((8, _E), jnp.int32),
        ],
        scratch_shapes=[
            pltpu.VMEM((8, _E), jnp.float32),
            pltpu.VMEM((_T, _T), jnp.float32),
        ],
        compiler_params=pltpu.CompilerParams(
            dimension_semantics=("arbitrary",)),
    )(x, bias8, wt)


@functools.lru_cache(maxsize=1)
def _sc_scatter_kernel():
    mesh = plsc.VectorSubcoreMesh(core_axis_name="c", subcore_axis_name="s",
                                  num_cores=_NC, num_subcores=_NS)
    return functools.partial(
        pl.kernel,
        mesh=mesh,
        out_type=[
            jax.ShapeDtypeStruct((_N * _K,), jnp.float32),
            jax.ShapeDtypeStruct((_N * _K,), jnp.int32),
        ],
        scratch_types=[
            pltpu.VMEM((_ROWS, _COLS), jnp.float32),
            pltpu.VMEM((_ROWS, _COLS), jnp.int32),
            pltpu.VMEM((_ROWS, _COLS), jnp.int32),
            pltpu.VMEM((_ROWS, _COLS), jnp.int32),
            pltpu.VMEM((_E,), jnp.int32),
            pltpu.SemaphoreType.DMA,
            pltpu.SemaphoreType.DMA,
        ],
        compiler_params=pltpu.CompilerParams(needs_layout_passes=False),
    )(_sc_scatter_body)


def _sc_scatter_body(vals_hbm, sel_hbm, posng_hbm, offs_hbm,
                out_s_hbm, out_t_hbm,
                val_v, sel_v, pos_v, tok_v, offs_v, sem_s, sem_t):
    wid = lax.axis_index("s") * _NC + lax.axis_index("c")
    row0 = wid * _ROWS
    base = wid * _S

    pltpu.sync_copy(vals_hbm.at[pl.ds(row0, _ROWS), :], val_v)
    pltpu.sync_copy(sel_hbm.at[pl.ds(row0, _ROWS), :], sel_v)
    pltpu.sync_copy(posng_hbm.at[pl.ds(row0, _ROWS), :], pos_v)
    pltpu.sync_copy(offs_hbm, offs_v)

    lane16 = lax.iota(jnp.int32, 16)

    def _chunk(j, _):
        r = j // 8
        c16 = (j % 8) * 16
        e = sel_v[r, pl.ds(c16, 16)]
        off = plsc.load_gather(offs_v, [e])
        pos_v[r, pl.ds(c16, 16)] = pos_v[r, pl.ds(c16, 16)] + off
        flat = base + r * _COLS + c16 + lane16
        tok_v[r, pl.ds(c16, 16)] = lax.shift_right_logical(flat, 3)
        return 0

    lax.fori_loop(0, _ROWS * 8, _chunk, 0)

    def _scat(j, _):
        pltpu.async_copy(val_v.at[j], out_s_hbm.at[pos_v.at[j]], sem_s)
        pltpu.async_copy(tok_v.at[j], out_t_hbm.at[pos_v.at[j]], sem_t)
        return 0

    lax.fori_loop(0, _ROWS, _scat, 0)
    pltpu.make_async_copy(vals_hbm.at[pl.ds(row0, _ROWS), :], val_v,
                          sem_s).wait()
    pltpu.make_async_copy(posng_hbm.at[pl.ds(row0, _ROWS), :], tok_v,
                          sem_t).wait()


def kernel(x, expert_bias, W):
    bias8 = jnp.broadcast_to(expert_bias.reshape(1, _E), (8, _E))
    wt = W.T
    vals, sel, posng, counts8, offs8 = _tc_route(x, bias8, wt)
    vals2 = vals.reshape(_NW * _ROWS, _COLS)
    sel2 = sel.reshape(_NW * _ROWS, _COLS)
    posng2 = posng.reshape(_NW * _ROWS, _COLS)
    offs = offs8[0]
    out_s, out_t = _sc_scatter_kernel()(vals2, sel2, posng2, offs)
    return out_s, out_t, counts8[0]


# speedup vs baseline: 1.0123x; 1.0093x over previous
import functools

import jax
import jax.numpy as jnp
from jax import lax
from jax.experimental import pallas as pl
from jax.experimental.pallas import tpu as pltpu
from jax.experimental.pallas import tpu_sc as plsc

_DIM = 768
_E = 64
_K = 8
_N = 32768
_T = 256
_NB = _N // _T

_NC = 2
_NS = 16
_NW = _NC * _NS
_S = _N * _K // _NW
_COLS = 128
_ROWS = _S // _COLS


def _tc_body(x_ref, bias_ref, wt_ref, vals_ref, sel_ref, posng_ref,
             counts_ref, offs_ref, carry, ltri):
    pid = pl.program_id(0)

    @pl.when(pid == 0)
    def _init():
        carry[...] = jnp.zeros_like(carry)
        r = lax.broadcasted_iota(jnp.int32, (_T, _T), 0)
        c = lax.broadcasted_iota(jnp.int32, (_T, _T), 1)
        ltri[...] = (c < r).astype(jnp.float32)

    x = x_ref[...]
    wt = wt_ref[...]
    logits = jnp.dot(x, wt, preferred_element_type=jnp.float32)
    m = jnp.max(logits, axis=1, keepdims=True)
    ex = jnp.exp(logits - m)
    probs = ex / jnp.sum(ex, axis=1, keepdims=True)

    bias = bias_ref[0:1, :]
    work = probs + bias
    lane = lax.broadcasted_iota(jnp.int32, (_T, _E), 1)
    hist = jnp.zeros((_T, _E), jnp.float32)
    sel_cols = []
    val_cols = []
    for _ in range(_K):
        mk = jnp.max(work, axis=1, keepdims=True)
        cand = jnp.where(work == mk, lane, _E)
        idx = jnp.min(cand, axis=1, keepdims=True)
        hot = lane == idx
        sel_cols.append(idx)
        val_cols.append(jnp.sum(jnp.where(hot, probs, 0.0), axis=1,
                                keepdims=True))
        work = jnp.where(hot, -jnp.inf, work)
        hist = hist + hot.astype(jnp.float32)

    vals_ref[...] = jnp.concatenate(val_cols, axis=1)
    sel_ref[...] = jnp.concatenate(sel_cols, axis=1)

    cum = jnp.dot(ltri[...], hist, preferred_element_type=jnp.float32)
    pos_tok = carry[0:1, :] + cum
    png_cols = [jnp.sum(jnp.where(lane == s, pos_tok, 0.0), axis=1,
                        keepdims=True) for s in sel_cols]
    posng_ref[...] = jnp.concatenate(png_cols, axis=1).astype(jnp.int32)

    new_carry = carry[0:1, :] + jnp.sum(hist, axis=0, keepdims=True)
    carry[...] = jnp.broadcast_to(new_carry, carry.shape)

    counts_ref[...] = jnp.broadcast_to(new_carry, (8, _E)).astype(jnp.int32)
    lo = lax.broadcasted_iota(jnp.int32, (_E, _E), 0)
    hi = lax.broadcasted_iota(jnp.int32, (_E, _E), 1)
    strict = (lo < hi).astype(jnp.float32)
    offs = jnp.dot(new_carry, strict, preferred_element_type=jnp.float32,
                   precision=lax.Precision.HIGHEST)
    offs_ref[...] = jnp.broadcast_to(offs, (8, _E)).astype(jnp.int32)


def _tc_route(x, bias8, wt):
    return pl.pallas_call(
        _tc_body,
        grid=(_NB,),
        in_specs=[
            pl.BlockSpec((_T, _DIM), lambda i: (i, 0)),
            pl.BlockSpec((8, _E), lambda i: (0, 0)),
            pl.BlockSpec((_DIM, _E), lambda i: (0, 0)),
        ],
        out_specs=[
            pl.BlockSpec((_T, _K), lambda i: (i, 0)),
            pl.BlockSpec((_T, _K), lambda i: (i, 0)),
            pl.BlockSpec((_T, _K), lambda i: (i, 0)),
            pl.BlockSpec((8, _E), lambda i: (0, 0)),
            pl.BlockSpec((8, _E), lambda i: (0, 0)),
        ],
        out_shape=[
            jax.ShapeDtypeStruct((_N, _K), jnp.float32),
            jax.ShapeDtypeStruct((_N, _K), jnp.int32),
            jax.ShapeDtypeStruct((_N, _K), jnp.int32),
            jax.ShapeDtypeStruct((8, _E), jnp.int32),
            jax.ShapeDtypeStruct((8, _E), jnp.int32),
        ],
        scratch_shapes=[
            pltpu.VMEM((8, _E), jnp.float32),
            pltpu.VMEM((_T, _T), jnp.float32),
        ],
        compiler_params=pltpu.CompilerParams(
            dimension_semantics=("arbitrary",)),
    )(x, bias8, wt)


@functools.lru_cache(maxsize=1)
def _sc_scatter_kernel():
    mesh = plsc.VectorSubcoreMesh(core_axis_name="c", subcore_axis_name="s",
                                  num_cores=_NC, num_subcores=_NS)
    return functools.partial(
        pl.kernel,
        mesh=mesh,
        out_type=[
            jax.ShapeDtypeStruct((_N * _K,), jnp.float32),
            jax.ShapeDtypeStruct((_N * _K,), jnp.int32),
        ],
        scratch_types=[
            pltpu.VMEM((_ROWS, _COLS), jnp.float32),
            pltpu.VMEM((_ROWS, _COLS), jnp.int32),
            pltpu.VMEM((_ROWS, _COLS), jnp.int32),
            pltpu.VMEM((_ROWS, _COLS), jnp.int32),
            pltpu.VMEM((_E,), jnp.int32),
            pltpu.SemaphoreType.DMA,
            pltpu.SemaphoreType.DMA,
        ],
        compiler_params=pltpu.CompilerParams(needs_layout_passes=False),
    )(_sc_scatter_body)


def _sc_scatter_body(vals_hbm, sel_hbm, posng_hbm, offs_hbm,
                out_s_hbm, out_t_hbm,
                val_v, sel_v, pos_v, tok_v, offs_v, sem_s, sem_t):
    wid = lax.axis_index("s") * _NC + lax.axis_index("c")
    row0 = wid * _ROWS
    base = wid * _S

    pltpu.async_copy(vals_hbm.at[pl.ds(row0, _ROWS), :], val_v, sem_s)
    pltpu.async_copy(sel_hbm.at[pl.ds(row0, _ROWS), :], sel_v, sem_t)
    pltpu.async_copy(posng_hbm.at[pl.ds(row0, _ROWS), :], pos_v, sem_s)
    pltpu.sync_copy(offs_hbm, offs_v)
    pltpu.make_async_copy(vals_hbm.at[pl.ds(row0, _ROWS), :], val_v,
                          sem_s).wait()
    pltpu.make_async_copy(sel_hbm.at[pl.ds(row0, _ROWS), :], sel_v,
                          sem_t).wait()
    pltpu.make_async_copy(posng_hbm.at[pl.ds(row0, _ROWS), :], pos_v,
                          sem_s).wait()

    lane16 = lax.iota(jnp.int32, 16)

    def _row(r, _):
        flat0 = base + r * _COLS + lane16
        for c in range(_COLS // 16):
            sl = pl.ds(c * 16, 16)
            e = sel_v[r, sl]
            off = plsc.load_gather(offs_v, [e])
            pos_v[r, sl] = pos_v[r, sl] + off
            tok_v[r, sl] = lax.shift_right_logical(flat0 + c * 16, 3)
        pltpu.async_copy(val_v.at[r], out_s_hbm.at[pos_v.at[r]], sem_s)
        pltpu.async_copy(tok_v.at[r], out_t_hbm.at[pos_v.at[r]], sem_t)
        return 0

    lax.fori_loop(0, _ROWS, _row, 0)
    pltpu.make_async_copy(vals_hbm.at[pl.ds(row0, _ROWS), :], val_v,
                          sem_s).wait()
    pltpu.make_async_copy(posng_hbm.at[pl.ds(row0, _ROWS), :], tok_v,
                          sem_t).wait()


def kernel(x, expert_bias, W):
    bias8 = jnp.broadcast_to(expert_bias.reshape(1, _E), (8, _E))
    wt = W.T
    vals, sel, posng, counts8, offs8 = _tc_route(x, bias8, wt)
    vals2 = vals.reshape(_NW * _ROWS, _COLS)
    sel2 = sel.reshape(_NW * _ROWS, _COLS)
    posng2 = posng.reshape(_NW * _ROWS, _COLS)
    offs = offs8[0]
    out_s, out_t = _sc_scatter_kernel()(vals2, sel2, posng2, offs)
    return out_s, out_t, counts8[0]


# speedup vs baseline: 2.1904x; 2.1638x over previous
import functools

import jax
import jax.numpy as jnp
from jax import lax
from jax.experimental import pallas as pl
from jax.experimental.pallas import tpu as pltpu
from jax.experimental.pallas import tpu_sc as plsc

_DIM = 768
_E = 64
_K = 8
_N = 32768
_T = 256
_NB = _N // _T

_NC = 2
_NS = 16
_NW = _NC * _NS
_S = _N * _K // _NW
_COLS = 128
_ROWS = _S // _COLS
_TROWS = _N * _K // _NS // _COLS
_HALF = _N * _K // _NC


def _tc_body(x_ref, bias_ref, wt_ref, vals_ref, sel_ref, posng_ref,
             counts_ref, offs_ref, carry, ltri):
    pid = pl.program_id(0)

    @pl.when(pid == 0)
    def _init():
        carry[...] = jnp.zeros_like(carry)
        r = lax.broadcasted_iota(jnp.int32, (_T, _T), 0)
        c = lax.broadcasted_iota(jnp.int32, (_T, _T), 1)
        ltri[...] = (c < r).astype(jnp.float32)

    x = x_ref[...]
    wt = wt_ref[...]
    logits = jnp.dot(x, wt, preferred_element_type=jnp.float32)
    m = jnp.max(logits, axis=1, keepdims=True)
    ex = jnp.exp(logits - m)
    probs = ex / jnp.sum(ex, axis=1, keepdims=True)

    bias = bias_ref[0:1, :]
    work = probs + bias
    lane = lax.broadcasted_iota(jnp.int32, (_T, _E), 1)
    hist = jnp.zeros((_T, _E), jnp.float32)
    sel_cols = []
    val_cols = []
    for _ in range(_K):
        mk = jnp.max(work, axis=1, keepdims=True)
        cand = jnp.where(work == mk, lane, _E)
        idx = jnp.min(cand, axis=1, keepdims=True)
        hot = lane == idx
        sel_cols.append(idx)
        val_cols.append(jnp.sum(jnp.where(hot, probs, 0.0), axis=1,
                                keepdims=True))
        work = jnp.where(hot, -jnp.inf, work)
        hist = hist + hot.astype(jnp.float32)

    vals_ref[...] = jnp.concatenate(val_cols, axis=1)
    sel_ref[...] = jnp.concatenate(sel_cols, axis=1)

    cum = jnp.dot(ltri[...], hist, preferred_element_type=jnp.float32)
    pos_tok = carry[0:1, :] + cum
    png_cols = [jnp.sum(jnp.where(lane == s, pos_tok, 0.0), axis=1,
                        keepdims=True) for s in sel_cols]
    posng_ref[...] = jnp.concatenate(png_cols, axis=1).astype(jnp.int32)

    new_carry = carry[0:1, :] + jnp.sum(hist, axis=0, keepdims=True)
    carry[...] = jnp.broadcast_to(new_carry, carry.shape)

    counts_ref[...] = jnp.broadcast_to(new_carry, (8, _E)).astype(jnp.int32)
    lo = lax.broadcasted_iota(jnp.int32, (_E, _E), 0)
    hi = lax.broadcasted_iota(jnp.int32, (_E, _E), 1)
    strict = (lo < hi).astype(jnp.float32)
    offs = jnp.dot(new_carry, strict, preferred_element_type=jnp.float32,
                   precision=lax.Precision.HIGHEST)
    offs_ref[...] = jnp.broadcast_to(offs, (8, _E)).astype(jnp.int32)


def _tc_route(x, bias8, wt):
    return pl.pallas_call(
        _tc_body,
        grid=(_NB,),
        in_specs=[
            pl.BlockSpec((_T, _DIM), lambda i: (i, 0)),
            pl.BlockSpec((8, _E), lambda i: (0, 0)),
            pl.BlockSpec((_DIM, _E), lambda i: (0, 0)),
        ],
        out_specs=[
            pl.BlockSpec((_T, _K), lambda i: (i, 0)),
            pl.BlockSpec((_T, _K), lambda i: (i, 0)),
            pl.BlockSpec((_T, _K), lambda i: (i, 0)),
            pl.BlockSpec((8, _E), lambda i: (0, 0)),
            pl.BlockSpec((8, _E), lambda i: (0, 0)),
        ],
        out_shape=[
            jax.ShapeDtypeStruct((_N, _K), jnp.float32),
            jax.ShapeDtypeStruct((_N, _K), jnp.int32),
            jax.ShapeDtypeStruct((_N, _K), jnp.int32),
            jax.ShapeDtypeStruct((8, _E), jnp.int32),
            jax.ShapeDtypeStruct((8, _E), jnp.int32),
        ],
        scratch_shapes=[
            pltpu.VMEM((8, _E), jnp.float32),
            pltpu.VMEM((_T, _T), jnp.float32),
        ],
        compiler_params=pltpu.CompilerParams(
            dimension_semantics=("arbitrary",)),
    )(x, bias8, wt)


@functools.lru_cache(maxsize=1)
def _sc_scatter_kernel():
    mesh = plsc.VectorSubcoreMesh(core_axis_name="c", subcore_axis_name="s",
                                  num_cores=_NC, num_subcores=_NS)
    return functools.partial(
        pl.kernel,
        mesh=mesh,
        out_type=[
            jax.ShapeDtypeStruct((_N * _K,), jnp.float32),
            jax.ShapeDtypeStruct((_N * _K,), jnp.int32),
        ],
        scratch_types=[
            pltpu.VMEM((_TROWS, _COLS), jnp.float32),
            pltpu.VMEM((_TROWS, _COLS), jnp.int32),
            pltpu.VMEM((_TROWS, _COLS), jnp.int32),
            pltpu.VMEM((_TROWS, _COLS), jnp.int32),
            pltpu.VMEM((_E,), jnp.int32),
            pltpu.VMEM((_HALF // _NS,), jnp.float32),
            pltpu.VMEM((_HALF // _NS,), jnp.int32),
            pltpu.VMEM_SHARED((_N * _K,), jnp.float32),
            pltpu.VMEM_SHARED((_N * _K,), jnp.int32),
            pltpu.SemaphoreType.DMA,
            pltpu.SemaphoreType.DMA,
        ],
        compiler_params=pltpu.CompilerParams(needs_layout_passes=False),
    )(_sc_scatter_body)


def _sc_scatter_body(vals_hbm, sel_hbm, posng_hbm, offs_hbm,
                     out_s_hbm, out_t_hbm,
                     val_v, sel_v, pos_v, tok_v, offs_v, bnc_s, bnc_t,
                     sp_s, sp_t, sem_s, sem_t):
    cid = lax.axis_index("c")
    sid = lax.axis_index("s")
    row0 = sid * _TROWS
    base = row0 * _COLS

    pltpu.async_copy(vals_hbm.at[pl.ds(row0, _TROWS), :], val_v, sem_s)
    pltpu.async_copy(sel_hbm.at[pl.ds(row0, _TROWS), :], sel_v, sem_t)
    pltpu.async_copy(posng_hbm.at[pl.ds(row0, _TROWS), :], pos_v, sem_s)
    pltpu.sync_copy(offs_hbm, offs_v)
    pltpu.make_async_copy(vals_hbm.at[pl.ds(row0, _TROWS), :], val_v,
                          sem_s).wait()
    pltpu.make_async_copy(sel_hbm.at[pl.ds(row0, _TROWS), :], sel_v,
                          sem_t).wait()
    pltpu.make_async_copy(posng_hbm.at[pl.ds(row0, _TROWS), :], pos_v,
                          sem_s).wait()

    lane16 = lax.iota(jnp.int32, 16)

    def _row(r, _):
        flat0 = base + r * _COLS + lane16
        for c in range(_COLS // 16):
            sl = pl.ds(c * 16, 16)
            e = sel_v[r, sl]
            off = plsc.load_gather(offs_v, [e])
            pos_v[r, sl] = pos_v[r, sl] + off
            tok_v[r, sl] = lax.shift_right_logical(flat0 + c * 16, 3)
        pltpu.async_copy(val_v.at[r], sp_s.at[pos_v.at[r]], sem_s)
        pltpu.async_copy(tok_v.at[r], sp_t.at[pos_v.at[r]], sem_t)
        return 0

    lax.fori_loop(0, _TROWS, _row, 0)
    pltpu.make_async_copy(vals_hbm.at[pl.ds(row0, _TROWS), :], val_v,
                          sem_s).wait()
    pltpu.make_async_copy(posng_hbm.at[pl.ds(row0, _TROWS), :], tok_v,
                          sem_t).wait()

    plsc.subcore_barrier()

    span = _HALF // _NS
    out0 = cid * _HALF + sid * span
    pltpu.sync_copy(sp_s.at[pl.ds(out0, span)], bnc_s)
    pltpu.sync_copy(sp_t.at[pl.ds(out0, span)], bnc_t)
    pltpu.async_copy(bnc_s, out_s_hbm.at[pl.ds(out0, span)], sem_s)
    pltpu.async_copy(bnc_t, out_t_hbm.at[pl.ds(out0, span)], sem_t)
    pltpu.make_async_copy(bnc_s, out_s_hbm.at[pl.ds(out0, span)],
                          sem_s).wait()
    pltpu.make_async_copy(bnc_t, out_t_hbm.at[pl.ds(out0, span)],
                          sem_t).wait()


def kernel(x, expert_bias, W):
    bias8 = jnp.broadcast_to(expert_bias.reshape(1, _E), (8, _E))
    wt = W.T
    vals, sel, posng, counts8, offs8 = _tc_route(x, bias8, wt)
    vals2 = vals.reshape(_NW * _ROWS, _COLS)
    sel2 = sel.reshape(_NW * _ROWS, _COLS)
    posng2 = posng.reshape(_NW * _ROWS, _COLS)
    offs = offs8[0]
    out_s, out_t = _sc_scatter_kernel()(vals2, sel2, posng2, offs)
    return out_s, out_t, counts8[0]


# speedup vs baseline: 2.6539x; 1.2116x over previous
import functools

import jax
import jax.numpy as jnp
from jax import lax
from jax.experimental import pallas as pl
from jax.experimental.pallas import tpu as pltpu
from jax.experimental.pallas import tpu_sc as plsc

_DIM = 768
_E = 64
_K = 8
_N = 32768
_T = 256
_NB = _N // _T

_NC = 2
_NS = 16
_NW = _NC * _NS
_S = _N * _K // _NW
_COLS = 128
_ROWS = _S // _COLS
_TROWS = _N * _K // _NS // _COLS
_HALF = _N * _K // _NC


def _tc_body(x_ref, bias_ref, wt_ref, vals_ref, sel_ref, posng_ref,
             counts_ref, offs_ref, carry, ltri):
    pid = pl.program_id(0)

    @pl.when(pid == 0)
    def _init():
        carry[...] = jnp.zeros_like(carry)
        r = lax.broadcasted_iota(jnp.int32, (_T, _T), 0)
        c = lax.broadcasted_iota(jnp.int32, (_T, _T), 1)
        ltri[...] = (c < r).astype(jnp.float32)

    x = x_ref[...]
    wt = wt_ref[...]
    logits = jnp.dot(x, wt, preferred_element_type=jnp.float32)
    m = jnp.max(logits, axis=1, keepdims=True)
    ex = jnp.exp(logits - m)
    probs = ex * (1.0 / jnp.sum(ex, axis=1, keepdims=True))

    bias = bias_ref[0:1, :]
    work = probs + bias
    lane = lax.broadcasted_iota(jnp.int32, (_T, _E), 1).astype(jnp.float32)
    hist = jnp.zeros((_T, _E), jnp.float32)
    sel_cols = []
    val_cols = []
    for _ in range(_K):
        mk = jnp.max(work, axis=1, keepdims=True)
        cand = jnp.where(work == mk, lane, float(_E))
        idx = jnp.min(cand, axis=1, keepdims=True)
        hot = lane == idx
        sel_cols.append(idx)
        val_cols.append(jnp.sum(jnp.where(hot, probs, 0.0), axis=1,
                                keepdims=True))
        work = jnp.where(hot, -jnp.inf, work)
        hist = hist + hot.astype(jnp.float32)

    vals_ref[...] = jnp.concatenate(val_cols, axis=1)
    sel_ref[...] = jnp.concatenate(sel_cols, axis=1).astype(jnp.int32)

    cum = jnp.dot(ltri[...], hist, preferred_element_type=jnp.float32)
    pos_tok = carry[0:1, :] + cum
    png_cols = [jnp.sum(jnp.where(lane == s, pos_tok, 0.0), axis=1,
                        keepdims=True) for s in sel_cols]
    posng_ref[...] = jnp.concatenate(png_cols, axis=1).astype(jnp.int32)

    new_carry = carry[0:1, :] + jnp.sum(hist, axis=0, keepdims=True)
    carry[...] = jnp.broadcast_to(new_carry, carry.shape)

    counts_ref[...] = jnp.broadcast_to(new_carry, (8, _E)).astype(jnp.int32)
    lo = lax.broadcasted_iota(jnp.int32, (_E, _E), 0)
    hi = lax.broadcasted_iota(jnp.int32, (_E, _E), 1)
    strict = (lo < hi).astype(jnp.float32)
    offs = jnp.dot(new_carry, strict, preferred_element_type=jnp.float32,
                   precision=lax.Precision.HIGHEST)
    offs_ref[...] = jnp.broadcast_to(offs, (8, _E)).astype(jnp.int32)


def _tc_route(x, bias8, wt):
    return pl.pallas_call(
        _tc_body,
        grid=(_NB,),
        in_specs=[
            pl.BlockSpec((_T, _DIM), lambda i: (i, 0)),
            pl.BlockSpec((8, _E), lambda i: (0, 0)),
            pl.BlockSpec((_DIM, _E), lambda i: (0, 0)),
        ],
        out_specs=[
            pl.BlockSpec((_T, _K), lambda i: (i, 0)),
            pl.BlockSpec((_T, _K), lambda i: (i, 0)),
            pl.BlockSpec((_T, _K), lambda i: (i, 0)),
            pl.BlockSpec((8, _E), lambda i: (0, 0)),
            pl.BlockSpec((8, _E), lambda i: (0, 0)),
        ],
        out_shape=[
            jax.ShapeDtypeStruct((_N, _K), jnp.float32),
            jax.ShapeDtypeStruct((_N, _K), jnp.int32),
            jax.ShapeDtypeStruct((_N, _K), jnp.int32),
            jax.ShapeDtypeStruct((8, _E), jnp.int32),
            jax.ShapeDtypeStruct((8, _E), jnp.int32),
        ],
        scratch_shapes=[
            pltpu.VMEM((8, _E), jnp.float32),
            pltpu.VMEM((_T, _T), jnp.float32),
        ],
        compiler_params=pltpu.CompilerParams(
            dimension_semantics=("arbitrary",)),
    )(x, bias8, wt)


@functools.lru_cache(maxsize=1)
def _sc_scatter_kernel():
    mesh = plsc.VectorSubcoreMesh(core_axis_name="c", subcore_axis_name="s",
                                  num_cores=_NC, num_subcores=_NS)
    return functools.partial(
        pl.kernel,
        mesh=mesh,
        out_type=[
            jax.ShapeDtypeStruct((_N * _K,), jnp.float32),
            jax.ShapeDtypeStruct((_N * _K,), jnp.int32),
        ],
        scratch_types=[
            pltpu.VMEM((_TROWS, _COLS), jnp.float32),
            pltpu.VMEM((_TROWS, _COLS), jnp.int32),
            pltpu.VMEM((_TROWS, _COLS), jnp.int32),
            pltpu.VMEM((_TROWS, _COLS), jnp.int32),
            pltpu.VMEM((_E,), jnp.int32),
            pltpu.VMEM((_HALF // _NS,), jnp.float32),
            pltpu.VMEM((_HALF // _NS,), jnp.int32),
            pltpu.VMEM_SHARED((_N * _K,), jnp.float32),
            pltpu.VMEM_SHARED((_N * _K,), jnp.int32),
            pltpu.SemaphoreType.DMA,
            pltpu.SemaphoreType.DMA,
        ],
        compiler_params=pltpu.CompilerParams(needs_layout_passes=False),
    )(_sc_scatter_body)


def _sc_scatter_body(vals_hbm, sel_hbm, posng_hbm, offs_hbm,
                     out_s_hbm, out_t_hbm,
                     val_v, sel_v, pos_v, tok_v, offs_v, bnc_s, bnc_t,
                     sp_s, sp_t, sem_s, sem_t):
    cid = lax.axis_index("c")
    sid = lax.axis_index("s")
    row0 = sid * _TROWS
    base = row0 * _COLS

    pltpu.async_copy(vals_hbm.at[pl.ds(row0, _TROWS), :], val_v, sem_s)
    pltpu.async_copy(sel_hbm.at[pl.ds(row0, _TROWS), :], sel_v, sem_t)
    pltpu.async_copy(posng_hbm.at[pl.ds(row0, _TROWS), :], pos_v, sem_s)
    pltpu.sync_copy(offs_hbm, offs_v)
    pltpu.make_async_copy(vals_hbm.at[pl.ds(row0, _TROWS), :], val_v,
                          sem_s).wait()
    pltpu.make_async_copy(sel_hbm.at[pl.ds(row0, _TROWS), :], sel_v,
                          sem_t).wait()
    pltpu.make_async_copy(posng_hbm.at[pl.ds(row0, _TROWS), :], pos_v,
                          sem_s).wait()

    lane16 = lax.iota(jnp.int32, 16)

    def _row(r, _):
        flat0 = base + r * _COLS + lane16
        for c in range(_COLS // 16):
            sl = pl.ds(c * 16, 16)
            e = sel_v[r, sl]
            off = plsc.load_gather(offs_v, [e])
            pos_v[r, sl] = pos_v[r, sl] + off
            tok_v[r, sl] = lax.shift_right_logical(flat0 + c * 16, 3)
        pltpu.async_copy(val_v.at[r], sp_s.at[pos_v.at[r]], sem_s)
        pltpu.async_copy(tok_v.at[r], sp_t.at[pos_v.at[r]], sem_t)
        return 0

    lax.fori_loop(0, _TROWS, _row, 0)
    pltpu.make_async_copy(vals_hbm.at[pl.ds(row0, _TROWS), :], val_v,
                          sem_s).wait()
    pltpu.make_async_copy(posng_hbm.at[pl.ds(row0, _TROWS), :], tok_v,
                          sem_t).wait()

    plsc.subcore_barrier()

    span = _HALF // _NS
    out0 = cid * _HALF + sid * span
    pltpu.sync_copy(sp_s.at[pl.ds(out0, span)], bnc_s)
    pltpu.sync_copy(sp_t.at[pl.ds(out0, span)], bnc_t)
    pltpu.async_copy(bnc_s, out_s_hbm.at[pl.ds(out0, span)], sem_s)
    pltpu.async_copy(bnc_t, out_t_hbm.at[pl.ds(out0, span)], sem_t)
    pltpu.make_async_copy(bnc_s, out_s_hbm.at[pl.ds(out0, span)],
                          sem_s).wait()
    pltpu.make_async_copy(bnc_t, out_t_hbm.at[pl.ds(out0, span)],
                          sem_t).wait()


def kernel(x, expert_bias, W):
    bias8 = jnp.broadcast_to(expert_bias.reshape(1, _E), (8, _E))
    wt = W.T
    vals, sel, posng, counts8, offs8 = _tc_route(x, bias8, wt)
    vals2 = vals.reshape(_NW * _ROWS, _COLS)
    sel2 = sel.reshape(_NW * _ROWS, _COLS)
    posng2 = posng.reshape(_NW * _ROWS, _COLS)
    offs = offs8[0]
    out_s, out_t = _sc_scatter_kernel()(vals2, sel2, posng2, offs)
    return out_s, out_t, counts8[0]


# speedup vs baseline: 3.5735x; 1.3465x over previous
import functools

import jax
import jax.numpy as jnp
from jax import lax
from jax.experimental import pallas as pl
from jax.experimental.pallas import tpu as pltpu
from jax.experimental.pallas import tpu_sc as plsc

_DIM = 768
_E = 64
_K = 8
_N = 32768
_T = 512
_NB = _N // _T

_NC = 2
_NS = 16
_NW = _NC * _NS
_S = _N * _K // _NW
_COLS = 128
_ROWS = _S // _COLS
_TROWS = _N * _K // _NS // _COLS
_HALF = _N * _K // _NC


def _tc_body(x_ref, bias_ref, wt_ref, vals_ref, sel_ref, posng_ref,
             counts_ref, offs_ref, carry, ltri):
    pid = pl.program_id(0)

    @pl.when(pid == 0)
    def _init():
        carry[...] = jnp.zeros_like(carry)
        r = lax.broadcasted_iota(jnp.int32, (_T, _T), 0)
        c = lax.broadcasted_iota(jnp.int32, (_T, _T), 1)
        ltri[...] = (c < r).astype(jnp.float32)

    x = x_ref[...]
    wt = wt_ref[...]
    logits = jnp.dot(x, wt, preferred_element_type=jnp.float32)
    m = jnp.max(logits, axis=1, keepdims=True)
    ex = jnp.exp(logits - m)
    probs = ex * (1.0 / jnp.sum(ex, axis=1, keepdims=True))

    bias = bias_ref[0:1, :]
    work = probs + bias
    lane = lax.broadcasted_iota(jnp.int32, (_T, _E), 1).astype(jnp.float32)
    hist = jnp.zeros((_T, _E), jnp.float32)
    sel_cols = []
    val_cols = []
    for _ in range(_K):
        mk = jnp.max(work, axis=1, keepdims=True)
        cand = jnp.where(work == mk, lane, float(_E))
        idx = jnp.min(cand, axis=1, keepdims=True)
        hot = lane == idx
        sel_cols.append(idx)
        val_cols.append(jnp.sum(jnp.where(hot, probs, 0.0), axis=1,
                                keepdims=True))
        work = jnp.where(hot, -jnp.inf, work)
        hist = hist + hot.astype(jnp.float32)

    vals_ref[...] = jnp.concatenate(val_cols, axis=1)
    sel_ref[...] = jnp.concatenate(sel_cols, axis=1).astype(jnp.int32)

    cum = jnp.dot(ltri[...], hist, preferred_element_type=jnp.float32)
    pos_tok = carry[0:1, :] + cum
    png_cols = [jnp.sum(jnp.where(lane == s, pos_tok, 0.0), axis=1,
                        keepdims=True) for s in sel_cols]
    posng_ref[...] = jnp.concatenate(png_cols, axis=1).astype(jnp.int32)

    new_carry = carry[0:1, :] + jnp.sum(hist, axis=0, keepdims=True)
    carry[...] = jnp.broadcast_to(new_carry, carry.shape)

    counts_ref[...] = jnp.broadcast_to(new_carry, (8, _E)).astype(jnp.int32)
    lo = lax.broadcasted_iota(jnp.int32, (_E, _E), 0)
    hi = lax.broadcasted_iota(jnp.int32, (_E, _E), 1)
    strict = (lo < hi).astype(jnp.float32)
    offs = jnp.dot(new_carry, strict, preferred_element_type=jnp.float32,
                   precision=lax.Precision.HIGHEST)
    offs_ref[...] = jnp.broadcast_to(offs, (8, _E)).astype(jnp.int32)


def _tc_route(x, bias8, wt):
    return pl.pallas_call(
        _tc_body,
        grid=(_NB,),
        in_specs=[
            pl.BlockSpec((_T, _DIM), lambda i: (i, 0)),
            pl.BlockSpec((8, _E), lambda i: (0, 0)),
            pl.BlockSpec((_DIM, _E), lambda i: (0, 0)),
        ],
        out_specs=[
            pl.BlockSpec((_T, _K), lambda i: (i, 0)),
            pl.BlockSpec((_T, _K), lambda i: (i, 0)),
            pl.BlockSpec((_T, _K), lambda i: (i, 0)),
            pl.BlockSpec((8, _E), lambda i: (0, 0)),
            pl.BlockSpec((8, _E), lambda i: (0, 0)),
        ],
        out_shape=[
            jax.ShapeDtypeStruct((_N, _K), jnp.float32),
            jax.ShapeDtypeStruct((_N, _K), jnp.int32),
            jax.ShapeDtypeStruct((_N, _K), jnp.int32),
            jax.ShapeDtypeStruct((8, _E), jnp.int32),
            jax.ShapeDtypeStruct((8, _E), jnp.int32),
        ],
        scratch_shapes=[
            pltpu.VMEM((8, _E), jnp.float32),
            pltpu.VMEM((_T, _T), jnp.float32),
        ],
        compiler_params=pltpu.CompilerParams(
            dimension_semantics=("arbitrary",)),
    )(x, bias8, wt)


@functools.lru_cache(maxsize=1)
def _sc_scatter_kernel():
    mesh = plsc.VectorSubcoreMesh(core_axis_name="c", subcore_axis_name="s",
                                  num_cores=_NC, num_subcores=_NS)
    return functools.partial(
        pl.kernel,
        mesh=mesh,
        out_type=[
            jax.ShapeDtypeStruct((_N * _K,), jnp.float32),
            jax.ShapeDtypeStruct((_N * _K,), jnp.int32),
        ],
        scratch_types=[
            pltpu.VMEM((_TROWS, _COLS), jnp.float32),
            pltpu.VMEM((_TROWS, _COLS), jnp.int32),
            pltpu.VMEM((_TROWS, _COLS), jnp.int32),
            pltpu.VMEM((_TROWS, _COLS), jnp.int32),
            pltpu.VMEM((_E,), jnp.int32),
            pltpu.VMEM((_HALF // _NS,), jnp.float32),
            pltpu.VMEM((_HALF // _NS,), jnp.int32),
            pltpu.VMEM_SHARED((_N * _K,), jnp.float32),
            pltpu.VMEM_SHARED((_N * _K,), jnp.int32),
            pltpu.SemaphoreType.DMA,
            pltpu.SemaphoreType.DMA,
        ],
        compiler_params=pltpu.CompilerParams(needs_layout_passes=False),
    )(_sc_scatter_body)


def _sc_scatter_body(vals_hbm, sel_hbm, posng_hbm, offs_hbm,
                     out_s_hbm, out_t_hbm,
                     val_v, sel_v, pos_v, tok_v, offs_v, bnc_s, bnc_t,
                     sp_s, sp_t, sem_s, sem_t):
    cid = lax.axis_index("c")
    sid = lax.axis_index("s")
    row0 = sid * _TROWS
    base = row0 * _COLS

    pltpu.async_copy(vals_hbm.at[pl.ds(row0, _TROWS), :], val_v, sem_s)
    pltpu.async_copy(sel_hbm.at[pl.ds(row0, _TROWS), :], sel_v, sem_t)
    pltpu.async_copy(posng_hbm.at[pl.ds(row0, _TROWS), :], pos_v, sem_s)
    pltpu.sync_copy(offs_hbm, offs_v)
    pltpu.make_async_copy(vals_hbm.at[pl.ds(row0, _TROWS), :], val_v,
                          sem_s).wait()
    pltpu.make_async_copy(sel_hbm.at[pl.ds(row0, _TROWS), :], sel_v,
                          sem_t).wait()
    pltpu.make_async_copy(posng_hbm.at[pl.ds(row0, _TROWS), :], pos_v,
                          sem_s).wait()

    lane16 = lax.iota(jnp.int32, 16)

    def _row(r, _):
        flat0 = base + r * _COLS + lane16
        for c in range(_COLS // 16):
            sl = pl.ds(c * 16, 16)
            e = sel_v[r, sl]
            off = plsc.load_gather(offs_v, [e])
            pos_v[r, sl] = pos_v[r, sl] + off
            tok_v[r, sl] = lax.shift_right_logical(flat0 + c * 16, 3)
        pltpu.async_copy(val_v.at[r], sp_s.at[pos_v.at[r]], sem_s)
        pltpu.async_copy(tok_v.at[r], sp_t.at[pos_v.at[r]], sem_t)
        return 0

    lax.fori_loop(0, _TROWS, _row, 0)
    pltpu.make_async_copy(vals_hbm.at[pl.ds(row0, _TROWS), :], val_v,
                          sem_s).wait()
    pltpu.make_async_copy(posng_hbm.at[pl.ds(row0, _TROWS), :], tok_v,
                          sem_t).wait()

    plsc.subcore_barrier()

    span = _HALF // _NS
    out0 = cid * _HALF + sid * span
    pltpu.sync_copy(sp_s.at[pl.ds(out0, span)], bnc_s)
    pltpu.sync_copy(sp_t.at[pl.ds(out0, span)], bnc_t)
    pltpu.async_copy(bnc_s, out_s_hbm.at[pl.ds(out0, span)], sem_s)
    pltpu.async_copy(bnc_t, out_t_hbm.at[pl.ds(out0, span)], sem_t)
    pltpu.make_async_copy(bnc_s, out_s_hbm.at[pl.ds(out0, span)],
                          sem_s).wait()
    pltpu.make_async_copy(bnc_t, out_t_hbm.at[pl.ds(out0, span)],
                          sem_t).wait()


def kernel(x, expert_bias, W):
    bias8 = jnp.broadcast_to(expert_bias.reshape(1, _E), (8, _E))
    wt = W.T
    vals, sel, posng, counts8, offs8 = _tc_route(x, bias8, wt)
    vals2 = vals.reshape(_NW * _ROWS, _COLS)
    sel2 = sel.reshape(_NW * _ROWS, _COLS)
    posng2 = posng.reshape(_NW * _ROWS, _COLS)
    offs = offs8[0]
    out_s, out_t = _sc_scatter_kernel()(vals2, sel2, posng2, offs)
    return out_s, out_t, counts8[0]


# speedup vs baseline: 3.6283x; 1.0153x over previous
import functools

import jax
import jax.numpy as jnp
from jax import lax
from jax.experimental import pallas as pl
from jax.experimental.pallas import tpu as pltpu
from jax.experimental.pallas import tpu_sc as plsc

_DIM = 768
_E = 64
_K = 8
_N = 32768
_T = 1024
_NB = _N // _T

_NC = 2
_NS = 16
_NW = _NC * _NS
_S = _N * _K // _NW
_COLS = 128
_ROWS = _S // _COLS
_TROWS = _N * _K // _NS // _COLS
_HALF = _N * _K // _NC


def _tc_body(x_ref, bias_ref, wt_ref, vals_ref, sel_ref, posng_ref,
             counts_ref, offs_ref, carry, ltri):
    pid = pl.program_id(0)

    @pl.when(pid == 0)
    def _init():
        carry[...] = jnp.zeros_like(carry)
        r = lax.broadcasted_iota(jnp.int32, (_T, _T), 0)
        c = lax.broadcasted_iota(jnp.int32, (_T, _T), 1)
        ltri[...] = (c < r).astype(jnp.float32)

    x = x_ref[...]
    wt = wt_ref[...]
    logits = jnp.dot(x, wt, preferred_element_type=jnp.float32)
    m = jnp.max(logits, axis=1, keepdims=True)
    ex = jnp.exp(logits - m)
    probs = ex * (1.0 / jnp.sum(ex, axis=1, keepdims=True))

    bias = bias_ref[0:1, :]
    work = probs + bias
    lane = lax.broadcasted_iota(jnp.int32, (_T, _E), 1).astype(jnp.float32)
    hist = jnp.zeros((_T, _E), jnp.float32)
    sel_cols = []
    val_cols = []
    for _ in range(_K):
        mk = jnp.max(work, axis=1, keepdims=True)
        cand = jnp.where(work == mk, lane, float(_E))
        idx = jnp.min(cand, axis=1, keepdims=True)
        hot = lane == idx
        sel_cols.append(idx)
        val_cols.append(jnp.sum(jnp.where(hot, probs, 0.0), axis=1,
                                keepdims=True))
        work = jnp.where(hot, -jnp.inf, work)
        hist = hist + hot.astype(jnp.float32)

    vals_ref[...] = jnp.concatenate(val_cols, axis=1)
    sel_ref[...] = jnp.concatenate(sel_cols, axis=1).astype(jnp.int32)

    cum = jnp.dot(ltri[...], hist, preferred_element_type=jnp.float32)
    pos_tok = carry[0:1, :] + cum
    png_cols = [jnp.sum(jnp.where(lane == s, pos_tok, 0.0), axis=1,
                        keepdims=True) for s in sel_cols]
    posng_ref[...] = jnp.concatenate(png_cols, axis=1).astype(jnp.int32)

    new_carry = carry[0:1, :] + jnp.sum(hist, axis=0, keepdims=True)
    carry[...] = jnp.broadcast_to(new_carry, carry.shape)

    counts_ref[...] = jnp.broadcast_to(new_carry, (8, _E)).astype(jnp.int32)
    lo = lax.broadcasted_iota(jnp.int32, (_E, _E), 0)
    hi = lax.broadcasted_iota(jnp.int32, (_E, _E), 1)
    strict = (lo < hi).astype(jnp.float32)
    offs = jnp.dot(new_carry, strict, preferred_element_type=jnp.float32,
                   precision=lax.Precision.HIGHEST)
    offs_ref[...] = jnp.broadcast_to(offs, (8, _E)).astype(jnp.int32)


def _tc_route(x, bias8, wt):
    return pl.pallas_call(
        _tc_body,
        grid=(_NB,),
        in_specs=[
            pl.BlockSpec((_T, _DIM), lambda i: (i, 0)),
            pl.BlockSpec((8, _E), lambda i: (0, 0)),
            pl.BlockSpec((_DIM, _E), lambda i: (0, 0)),
        ],
        out_specs=[
            pl.BlockSpec((_T, _K), lambda i: (i, 0)),
            pl.BlockSpec((_T, _K), lambda i: (i, 0)),
            pl.BlockSpec((_T, _K), lambda i: (i, 0)),
            pl.BlockSpec((8, _E), lambda i: (0, 0)),
            pl.BlockSpec((8, _E), lambda i: (0, 0)),
        ],
        out_shape=[
            jax.ShapeDtypeStruct((_N, _K), jnp.float32),
            jax.ShapeDtypeStruct((_N, _K), jnp.int32),
            jax.ShapeDtypeStruct((_N, _K), jnp.int32),
            jax.ShapeDtypeStruct((8, _E), jnp.int32),
            jax.ShapeDtypeStruct((8, _E), jnp.int32),
        ],
        scratch_shapes=[
            pltpu.VMEM((8, _E), jnp.float32),
            pltpu.VMEM((_T, _T), jnp.float32),
        ],
        compiler_params=pltpu.CompilerParams(
            dimension_semantics=("arbitrary",)),
    )(x, bias8, wt)


@functools.lru_cache(maxsize=1)
def _sc_scatter_kernel():
    mesh = plsc.VectorSubcoreMesh(core_axis_name="c", subcore_axis_name="s",
                                  num_cores=_NC, num_subcores=_NS)
    return functools.partial(
        pl.kernel,
        mesh=mesh,
        out_type=[
            jax.ShapeDtypeStruct((_N * _K,), jnp.float32),
            jax.ShapeDtypeStruct((_N * _K,), jnp.int32),
        ],
        scratch_types=[
            pltpu.VMEM((_TROWS, _COLS), jnp.float32),
            pltpu.VMEM((_TROWS, _COLS), jnp.int32),
            pltpu.VMEM((_TROWS, _COLS), jnp.int32),
            pltpu.VMEM((_TROWS, _COLS), jnp.int32),
            pltpu.VMEM((_E,), jnp.int32),
            pltpu.VMEM((_HALF // _NS,), jnp.float32),
            pltpu.VMEM((_HALF // _NS,), jnp.int32),
            pltpu.VMEM_SHARED((_N * _K,), jnp.float32),
            pltpu.VMEM_SHARED((_N * _K,), jnp.int32),
            pltpu.SemaphoreType.DMA,
            pltpu.SemaphoreType.DMA,
        ],
        compiler_params=pltpu.CompilerParams(needs_layout_passes=False),
    )(_sc_scatter_body)


def _sc_scatter_body(vals_hbm, sel_hbm, posng_hbm, offs_hbm,
                     out_s_hbm, out_t_hbm,
                     val_v, sel_v, pos_v, tok_v, offs_v, bnc_s, bnc_t,
                     sp_s, sp_t, sem_s, sem_t):
    cid = lax.axis_index("c")
    sid = lax.axis_index("s")
    row0 = sid * _TROWS
    base = row0 * _COLS

    pltpu.async_copy(vals_hbm.at[pl.ds(row0, _TROWS), :], val_v, sem_s)
    pltpu.async_copy(sel_hbm.at[pl.ds(row0, _TROWS), :], sel_v, sem_t)
    pltpu.async_copy(posng_hbm.at[pl.ds(row0, _TROWS), :], pos_v, sem_s)
    pltpu.sync_copy(offs_hbm, offs_v)
    pltpu.make_async_copy(vals_hbm.at[pl.ds(row0, _TROWS), :], val_v,
                          sem_s).wait()
    pltpu.make_async_copy(sel_hbm.at[pl.ds(row0, _TROWS), :], sel_v,
                          sem_t).wait()
    pltpu.make_async_copy(posng_hbm.at[pl.ds(row0, _TROWS), :], pos_v,
                          sem_s).wait()

    lane16 = lax.iota(jnp.int32, 16)

    def _row(r, _):
        flat0 = base + r * _COLS + lane16
        for c in range(_COLS // 16):
            sl = pl.ds(c * 16, 16)
            e = sel_v[r, sl]
            off = plsc.load_gather(offs_v, [e])
            pos_v[r, sl] = pos_v[r, sl] + off
            tok_v[r, sl] = lax.shift_right_logical(flat0 + c * 16, 3)
        pltpu.async_copy(val_v.at[r], sp_s.at[pos_v.at[r]], sem_s)
        pltpu.async_copy(tok_v.at[r], sp_t.at[pos_v.at[r]], sem_t)
        return 0

    lax.fori_loop(0, _TROWS, _row, 0)
    pltpu.make_async_copy(vals_hbm.at[pl.ds(row0, _TROWS), :], val_v,
                          sem_s).wait()
    pltpu.make_async_copy(posng_hbm.at[pl.ds(row0, _TROWS), :], tok_v,
                          sem_t).wait()

    plsc.subcore_barrier()

    span = _HALF // _NS
    out0 = cid * _HALF + sid * span
    pltpu.sync_copy(sp_s.at[pl.ds(out0, span)], bnc_s)
    pltpu.sync_copy(sp_t.at[pl.ds(out0, span)], bnc_t)
    pltpu.async_copy(bnc_s, out_s_hbm.at[pl.ds(out0, span)], sem_s)
    pltpu.async_copy(bnc_t, out_t_hbm.at[pl.ds(out0, span)], sem_t)
    pltpu.make_async_copy(bnc_s, out_s_hbm.at[pl.ds(out0, span)],
                          sem_s).wait()
    pltpu.make_async_copy(bnc_t, out_t_hbm.at[pl.ds(out0, span)],
                          sem_t).wait()


def kernel(x, expert_bias, W):
    bias8 = jnp.broadcast_to(expert_bias.reshape(1, _E), (8, _E))
    wt = W.T
    vals, sel, posng, counts8, offs8 = _tc_route(x, bias8, wt)
    vals2 = vals.reshape(_NW * _ROWS, _COLS)
    sel2 = sel.reshape(_NW * _ROWS, _COLS)
    posng2 = posng.reshape(_NW * _ROWS, _COLS)
    offs = offs8[0]
    out_s, out_t = _sc_scatter_kernel()(vals2, sel2, posng2, offs)
    return out_s, out_t, counts8[0]
